# baseline (speedup 1.0000x reference)
import functools

import jax
import jax.numpy as jnp
from jax import lax
from jax.experimental import pallas as pl
from jax.experimental.pallas import tpu as pltpu
from jax.experimental.pallas import tpu_sc as plsc

N_TOTAL = 16384
BS = 8192
HID = 256
NN = 100000
NC = 2
NS = 16
NW = NC * NS
L = 16

CHUNK = 6256
TAB = NS * CHUNK
RPT = BS // NW
RC = 128
NCH = RPT // RC
XC = 64
NXC = RPT // XC
IN_GROUPS = BS // L
TAB_GROUPS = CHUNK // L


def _body(x_hbm, nid_hbm, hist_hbm, out_hbm,
          inids_v, tab_v, outids_v, pos_v, xidx_v, sidx_v, hist_b,
          stage_b, tab_sh, sem_in0, sem_in1, sem_h, sem_w, sem_x, sem_s,
          sem_t):
    c = lax.axis_index("c")
    s = lax.axis_index("s")
    wid = c * NS + s
    row0 = wid * RPT
    iota = lax.iota(jnp.int32, L)

    scope = jax.named_scope
    for ch in range(NCH):
        pltpu.sync_copy(nid_hbm.at[pl.ds(BS + row0 + ch * RC, RC)],
                        outids_v.at[ch])
    hist_dmas = [
        pltpu.async_copy(hist_hbm.at[outids_v.at[ch]],
                         hist_b.at[pl.ds(ch * RC, RC)], sem_h)
        for ch in range(NCH)
    ]
    in_dmas = [
        pltpu.async_copy(nid_hbm.at[pl.ds(half * (BS // 2), BS // 2)],
                         inids_v.at[pl.ds(half * (BS // 2), BS // 2)],
                         sem)
        for half, sem in ((0, sem_in0), (1, sem_in1))
    ]
    tr0 = pltpu.async_copy(x_hbm.at[pl.ds(row0, RC)], stage_b, sem_t)

    neg1 = jnp.full((L,), -1, jnp.int32)
    _s1 = scope("p1_scan"); _s1.__enter__()

    def init_body(g, carry):
        for k in range(4):
            tab_v[pl.ds((g * 4 + k) * L, L)] = neg1
        return carry

    lax.fori_loop(0, TAB_GROUPS // 4, init_body, 0)
    for k in range(TAB_GROUPS - (TAB_GROUPS // 4) * 4):
        tab_v[pl.ds(((TAB_GROUPS // 4) * 4 + k) * L, L)] = neg1

    base = s * CHUNK

    def scan_one(g, acc):
        ids = inids_v[pl.ds(g * L, L)]
        li = ids - base
        inr = (li >= 0) & (li < CHUNK)
        j16 = g * L + iota
        lic = jnp.where(inr, li, 0)
        plsc.store_scatter(tab_v, [lic], j16, mask=inr)
        r1 = plsc.load_gather(tab_v, [lic])
        w1 = inr & (j16 > r1)
        plsc.store_scatter(tab_v, [lic], j16, mask=w1)
        return acc | (plsc.all_reduce_population_count(w1) >= 2)

    def scan_body(h, acc):
        for k in range(8):
            acc = scan_one(h * 8 + k, acc)
        return acc

    false16 = jnp.zeros((L,), jnp.bool_)
    in_dmas[0].wait()
    acc = lax.fori_loop(0, IN_GROUPS // 16, scan_body, false16)
    tr0.wait()
    tw0 = pltpu.async_copy(stage_b, out_hbm.at[pl.ds(row0, RC)], sem_t)
    in_dmas[1].wait()
    acc = lax.fori_loop(IN_GROUPS // 16, IN_GROUPS // 8, scan_body, acc)

    @pl.when(jnp.any(acc))
    def _serial_redo():
        def redo_body(g, carry):
            ids = inids_v[pl.ds(g * L, L)]
            li = ids - base
            inr = (li >= 0) & (li < CHUNK)
            j16 = g * L + iota
            lic = jnp.where(inr, li, 0)
            for k in range(L):
                plsc.store_scatter(tab_v, [lic], j16,
                                   mask=inr & (iota == k))
            return carry

        lax.fori_loop(0, IN_GROUPS, redo_body, 0)

    _s1.__exit__(None, None, None)
    _s2 = scope("p2_barrier"); _s2.__enter__()
    for dma in hist_dmas:
        dma.wait()
    w_dmas = [
        pltpu.async_copy(hist_b.at[pl.ds(ch * RC, RC)],
                         out_hbm.at[pl.ds(BS + row0 + ch * RC, RC)], sem_w)
        for ch in range(NCH)
    ]
    pltpu.sync_copy(tab_v, tab_sh.at[pl.ds(base, CHUNK)])
    plsc.subcore_barrier()
    _s2.__exit__(None, None, None)
    _s3 = scope("p3_pos"); _s3.__enter__()

    pos_dmas = [
        pltpu.async_copy(tab_sh.at[outids_v.at[ch]], pos_v.at[ch], sem_x)
        for ch in range(NCH)
    ]
    for dma in pos_dmas:
        dma.wait()
    for gg in range(RPT // L):
        pos16 = pos_v[gg // (RC // L), pl.ds((gg % (RC // L)) * L, L)]
        q16 = row0 + gg * L + iota
        m = pos16 >= 0
        xi = jnp.where(m, pos16, q16)
        si = jnp.where(m, BS + q16, q16)
        xidx_v[gg // (XC // L), pl.ds((gg % (XC // L)) * L, L)] = xi
        sidx_v[gg // (XC // L), pl.ds((gg % (XC // L)) * L, L)] = si

    tw0.wait()
    tr1 = pltpu.async_copy(x_hbm.at[pl.ds(row0 + RC, RC)], stage_b, sem_t)

    _s3.__exit__(None, None, None)
    _s4 = scope("p4_hist"); _s4.__enter__()
    _s4.__exit__(None, None, None)
    _s5 = scope("p5_ring"); _s5.__enter__()
    xg_sems = (sem_x, sem_in0, sem_in1, sem_h)
    xg = []
    for q in range(NXC):
        if q % 2 == 0:
            w_dmas[q // 2].wait()
        xg.append(
            pltpu.async_copy(x_hbm.at[xidx_v.at[q]],
                             hist_b.at[pl.ds(q * XC, XC)], xg_sems[q]))
    tr1.wait()
    tw1 = pltpu.async_copy(stage_b, out_hbm.at[pl.ds(row0 + RC, RC)],
                           sem_t)
    s_dmas = []
    for q in range(NXC):
        xg[q].wait()
        s_dmas.append(
            pltpu.async_copy(hist_b.at[pl.ds(q * XC, XC)],
                             out_hbm.at[sidx_v.at[q]], sem_s))
    for dma in s_dmas:
        dma.wait()
    tw1.wait()
    _s5.__exit__(None, None, None)


@functools.partial(jax.jit, static_argnums=())
def kernel(x, n_id, batch_size, hist_emb):
    del batch_size
    mesh = plsc.VectorSubcoreMesh(core_axis_name="c", subcore_axis_name="s")
    run = pl.kernel(
        _body,
        out_type=jax.ShapeDtypeStruct((N_TOTAL, HID), jnp.float32),
        mesh=mesh,
        compiler_params=pltpu.CompilerParams(needs_layout_passes=False),
        scratch_types=[
            pltpu.VMEM((BS,), jnp.int32),
            pltpu.VMEM((CHUNK,), jnp.int32),
            pltpu.VMEM((NCH, RC), jnp.int32),
            pltpu.VMEM((NCH, RC), jnp.int32),
            pltpu.VMEM((NXC, XC), jnp.int32),
            pltpu.VMEM((NXC, XC), jnp.int32),
            pltpu.VMEM((RPT, HID), jnp.float32),
            pltpu.VMEM((RC, HID), jnp.float32),
            pltpu.VMEM_SHARED((TAB,), jnp.int32),
            pltpu.SemaphoreType.DMA,
            pltpu.SemaphoreType.DMA,
            pltpu.SemaphoreType.DMA,
            pltpu.SemaphoreType.DMA,
            pltpu.SemaphoreType.DMA,
            pltpu.SemaphoreType.DMA,
            pltpu.SemaphoreType.DMA,
        ],
    )
    return run(x, n_id, hist_emb)


# speedup vs baseline: 1.0049x; 1.0049x over previous
import functools

import jax
import jax.numpy as jnp
from jax import lax
from jax.experimental import pallas as pl
from jax.experimental.pallas import tpu as pltpu
from jax.experimental.pallas import tpu_sc as plsc

N_TOTAL = 16384
BS = 8192
HID = 256
NN = 100000
NC = 2
NS = 16
NW = NC * NS
L = 16

CHUNK = 6256
TAB = NS * CHUNK
RPT = BS // NW
RC = 128
NCH = RPT // RC
XC = 64
NXC = RPT // XC
IN_GROUPS = BS // L
TAB_GROUPS = CHUNK // L


def _body(x_hbm, nid_hbm, hist_hbm, out_hbm,
          inids_v, tab_v, outids_v, pos_v, xidx_v, sidx_v, hist_b,
          stage_b, tab_sh, sem_in0, sem_in1, sem_h, sem_w, sem_x, sem_s,
          sem_t):
    c = lax.axis_index("c")
    s = lax.axis_index("s")
    wid = c * NS + s
    row0 = wid * RPT
    iota = lax.iota(jnp.int32, L)

    for ch in range(NCH):
        pltpu.sync_copy(nid_hbm.at[pl.ds(BS + row0 + ch * RC, RC)],
                        outids_v.at[ch])
    hist_dmas = [
        pltpu.async_copy(hist_hbm.at[outids_v.at[ch]],
                         hist_b.at[pl.ds(ch * RC, RC)], sem_h)
        for ch in range(NCH)
    ]
    in_dmas = [
        pltpu.async_copy(nid_hbm.at[pl.ds(half * (BS // 2), BS // 2)],
                         inids_v.at[pl.ds(half * (BS // 2), BS // 2)],
                         sem)
        for half, sem in ((0, sem_in0), (1, sem_in1))
    ]
    tr0 = pltpu.async_copy(x_hbm.at[pl.ds(row0, RC)], stage_b, sem_t)

    neg1 = jnp.full((L,), -1, jnp.int32)

    def init_body(g, carry):
        for k in range(4):
            tab_v[pl.ds((g * 4 + k) * L, L)] = neg1
        return carry

    lax.fori_loop(0, TAB_GROUPS // 4, init_body, 0)
    for k in range(TAB_GROUPS - (TAB_GROUPS // 4) * 4):
        tab_v[pl.ds(((TAB_GROUPS // 4) * 4 + k) * L, L)] = neg1

    base = s * CHUNK

    def scan_one(g, acc):
        ids = inids_v[pl.ds(g * L, L)]
        li = ids - base
        inr = (li >= 0) & (li < CHUNK)
        j16 = g * L + iota
        lic = jnp.where(inr, li, 0)
        plsc.store_scatter(tab_v, [lic], j16, mask=inr)
        r1 = plsc.load_gather(tab_v, [lic])
        w1 = inr & (j16 > r1)
        plsc.store_scatter(tab_v, [lic], j16, mask=w1)
        return acc | (plsc.all_reduce_population_count(w1) >= 2)

    def scan_body(h, acc):
        for k in range(8):
            acc = scan_one(h * 8 + k, acc)
        return acc

    false16 = jnp.zeros((L,), jnp.bool_)
    in_dmas[0].wait()
    acc = lax.fori_loop(0, IN_GROUPS // 16, scan_body, false16)
    tr0.wait()
    tw0 = pltpu.async_copy(stage_b, out_hbm.at[pl.ds(row0, RC)], sem_t)
    in_dmas[1].wait()
    acc = lax.fori_loop(IN_GROUPS // 16, IN_GROUPS // 8, scan_body, acc)

    @pl.when(jnp.any(acc))
    def _serial_redo():
        def redo_body(g, carry):
            ids = inids_v[pl.ds(g * L, L)]
            li = ids - base
            inr = (li >= 0) & (li < CHUNK)
            j16 = g * L + iota
            lic = jnp.where(inr, li, 0)
            for k in range(L):
                plsc.store_scatter(tab_v, [lic], j16,
                                   mask=inr & (iota == k))
            return carry

        lax.fori_loop(0, IN_GROUPS, redo_body, 0)

    for dma in hist_dmas:
        dma.wait()
    w_dmas = [
        pltpu.async_copy(hist_b.at[pl.ds(ch * RC, RC)],
                         out_hbm.at[pl.ds(BS + row0 + ch * RC, RC)], sem_w)
        for ch in range(NCH)
    ]
    pltpu.sync_copy(tab_v, tab_sh.at[pl.ds(base, CHUNK)])
    plsc.subcore_barrier()

    pos_dmas = [
        pltpu.async_copy(tab_sh.at[outids_v.at[ch]], pos_v.at[ch], sem_x)
        for ch in range(NCH)
    ]
    for dma in pos_dmas:
        dma.wait()
    for gg in range(RPT // L):
        pos16 = pos_v[gg // (RC // L), pl.ds((gg % (RC // L)) * L, L)]
        q16 = row0 + gg * L + iota
        m = pos16 >= 0
        xi = jnp.where(m, pos16, q16)
        si = jnp.where(m, BS + q16, q16)
        xidx_v[gg // (XC // L), pl.ds((gg % (XC // L)) * L, L)] = xi
        sidx_v[gg // (XC // L), pl.ds((gg % (XC // L)) * L, L)] = si

    tw0.wait()
    tr1 = pltpu.async_copy(x_hbm.at[pl.ds(row0 + RC, RC)], stage_b, sem_t)

    xg_sems = (sem_x, sem_in0, sem_in1, sem_h)
    xg = []
    for q in range(NXC):
        if q % 2 == 0:
            w_dmas[q // 2].wait()
        xg.append(
            pltpu.async_copy(x_hbm.at[xidx_v.at[q]],
                             hist_b.at[pl.ds(q * XC, XC)], xg_sems[q]))
    tr1.wait()
    tw1 = pltpu.async_copy(stage_b, out_hbm.at[pl.ds(row0 + RC, RC)],
                           sem_t)
    s_dmas = []
    for q in range(NXC):
        xg[q].wait()
        s_dmas.append(
            pltpu.async_copy(hist_b.at[pl.ds(q * XC, XC)],
                             out_hbm.at[sidx_v.at[q]], sem_s))
    for dma in s_dmas:
        dma.wait()
    tw1.wait()


@functools.partial(jax.jit, static_argnums=())
def kernel(x, n_id, batch_size, hist_emb):
    del batch_size
    mesh = plsc.VectorSubcoreMesh(core_axis_name="c", subcore_axis_name="s")
    run = pl.kernel(
        _body,
        out_type=jax.ShapeDtypeStruct((N_TOTAL, HID), jnp.float32),
        mesh=mesh,
        compiler_params=pltpu.CompilerParams(needs_layout_passes=False),
        scratch_types=[
            pltpu.VMEM((BS,), jnp.int32),
            pltpu.VMEM((CHUNK,), jnp.int32),
            pltpu.VMEM((NCH, RC), jnp.int32),
            pltpu.VMEM((NCH, RC), jnp.int32),
            pltpu.VMEM((NXC, XC), jnp.int32),
            pltpu.VMEM((NXC, XC), jnp.int32),
            pltpu.VMEM((RPT, HID), jnp.float32),
            pltpu.VMEM((RC, HID), jnp.float32),
            pltpu.VMEM_SHARED((TAB,), jnp.int32),
            pltpu.SemaphoreType.DMA,
            pltpu.SemaphoreType.DMA,
            pltpu.SemaphoreType.DMA,
            pltpu.SemaphoreType.DMA,
            pltpu.SemaphoreType.DMA,
            pltpu.SemaphoreType.DMA,
            pltpu.SemaphoreType.DMA,
        ],
    )
    return run(x, n_id, hist_emb)


# speedup vs baseline: 1.0130x; 1.0081x over previous
import functools

import jax
import jax.numpy as jnp
from jax import lax
from jax.experimental import pallas as pl
from jax.experimental.pallas import tpu as pltpu
from jax.experimental.pallas import tpu_sc as plsc

N_TOTAL = 16384
BS = 8192
HID = 256
NN = 100000
NC = 2
NS = 16
NW = NC * NS
L = 16

CHUNK = 6256
TAB = NS * CHUNK
RPT = BS // NW
RC = 128
NCH = RPT // RC
XC = 64
NXC = RPT // XC
IN_GROUPS = BS // L
TAB_GROUPS = CHUNK // L


def _body(x_hbm, nid_hbm, hist_hbm, out_hbm,
          inids_v, tab_v, outids_v, pos_v, xidx_v, sidx_v, hist_b,
          stage_b, tab_sh, sem_in0, sem_in1, sem_h, sem_w, sem_x, sem_s,
          sem_t):
    c = lax.axis_index("c")
    s = lax.axis_index("s")
    wid = c * NS + s
    row0 = wid * RPT
    iota = lax.iota(jnp.int32, L)

    oid_dmas = [
        pltpu.async_copy(nid_hbm.at[pl.ds(BS + row0 + ch * RC, RC)],
                         outids_v.at[ch], sem)
        for ch, sem in ((0, sem_w), (1, sem_s))
    ]
    for dma in oid_dmas:
        dma.wait()
    hist_dmas = [
        pltpu.async_copy(hist_hbm.at[outids_v.at[ch]],
                         hist_b.at[pl.ds(ch * RC, RC)], sem_h)
        for ch in range(NCH)
    ]
    in_dmas = [
        pltpu.async_copy(nid_hbm.at[pl.ds(half * (BS // 2), BS // 2)],
                         inids_v.at[pl.ds(half * (BS // 2), BS // 2)],
                         sem)
        for half, sem in ((0, sem_in0), (1, sem_in1))
    ]
    tr0 = pltpu.async_copy(x_hbm.at[pl.ds(row0, RC)], stage_b, sem_t)

    neg1 = jnp.full((L,), -1, jnp.int32)

    def init_body(g, carry):
        for k in range(4):
            tab_v[pl.ds((g * 4 + k) * L, L)] = neg1
        return carry

    lax.fori_loop(0, TAB_GROUPS // 4, init_body, 0)
    for k in range(TAB_GROUPS - (TAB_GROUPS // 4) * 4):
        tab_v[pl.ds(((TAB_GROUPS // 4) * 4 + k) * L, L)] = neg1

    base = s * CHUNK

    def scan_one(g, acc):
        ids = inids_v[pl.ds(g * L, L)]
        li = ids - base
        inr = (li >= 0) & (li < CHUNK)
        j16 = g * L + iota
        lic = jnp.where(inr, li, 0)
        plsc.store_scatter(tab_v, [lic], j16, mask=inr)
        r1 = plsc.load_gather(tab_v, [lic])
        w1 = inr & (j16 > r1)
        plsc.store_scatter(tab_v, [lic], j16, mask=w1)
        return acc | (plsc.all_reduce_population_count(w1) >= 2)

    def scan_body(h, acc):
        for k in range(8):
            acc = scan_one(h * 8 + k, acc)
        return acc

    false16 = jnp.zeros((L,), jnp.bool_)
    in_dmas[0].wait()
    acc = lax.fori_loop(0, IN_GROUPS // 16, scan_body, false16)
    tr0.wait()
    tw0 = pltpu.async_copy(stage_b, out_hbm.at[pl.ds(row0, RC)], sem_t)
    in_dmas[1].wait()
    acc = lax.fori_loop(IN_GROUPS // 16, IN_GROUPS // 8, scan_body, acc)

    @pl.when(jnp.any(acc))
    def _serial_redo():
        def redo_body(g, carry):
            ids = inids_v[pl.ds(g * L, L)]
            li = ids - base
            inr = (li >= 0) & (li < CHUNK)
            j16 = g * L + iota
            lic = jnp.where(inr, li, 0)
            for k in range(L):
                plsc.store_scatter(tab_v, [lic], j16,
                                   mask=inr & (iota == k))
            return carry

        lax.fori_loop(0, IN_GROUPS, redo_body, 0)

    for dma in hist_dmas:
        dma.wait()
    w_dmas = [
        pltpu.async_copy(hist_b.at[pl.ds(ch * RC, RC)],
                         out_hbm.at[pl.ds(BS + row0 + ch * RC, RC)], sem_w)
        for ch in range(NCH)
    ]
    pltpu.sync_copy(tab_v, tab_sh.at[pl.ds(base, CHUNK)])
    plsc.subcore_barrier()

    tw0.wait()
    tr1 = pltpu.async_copy(x_hbm.at[pl.ds(row0 + RC, RC)], stage_b, sem_t)

    pos_dmas = [
        pltpu.async_copy(tab_sh.at[outids_v.at[ch]], pos_v.at[ch], sem_x)
        for ch in range(NCH)
    ]
    for dma in pos_dmas:
        dma.wait()
    for gg in range(RPT // L):
        pos16 = pos_v[gg // (RC // L), pl.ds((gg % (RC // L)) * L, L)]
        q16 = row0 + gg * L + iota
        m = pos16 >= 0
        xi = jnp.where(m, pos16, q16)
        si = jnp.where(m, BS + q16, q16)
        xidx_v[gg // (XC // L), pl.ds((gg % (XC // L)) * L, L)] = xi
        sidx_v[gg // (XC // L), pl.ds((gg % (XC // L)) * L, L)] = si

    xg_sems = (sem_x, sem_in0, sem_in1, sem_h)
    xg = []
    for q in range(NXC):
        if q % 2 == 0:
            w_dmas[q // 2].wait()
        xg.append(
            pltpu.async_copy(x_hbm.at[xidx_v.at[q]],
                             hist_b.at[pl.ds(q * XC, XC)], xg_sems[q]))
    tr1.wait()
    tw1 = pltpu.async_copy(stage_b, out_hbm.at[pl.ds(row0 + RC, RC)],
                           sem_t)
    s_dmas = []
    for q in range(NXC):
        xg[q].wait()
        s_dmas.append(
            pltpu.async_copy(hist_b.at[pl.ds(q * XC, XC)],
                             out_hbm.at[sidx_v.at[q]], sem_s))
    for dma in s_dmas:
        dma.wait()
    tw1.wait()


@functools.partial(jax.jit, static_argnums=())
def kernel(x, n_id, batch_size, hist_emb):
    del batch_size
    mesh = plsc.VectorSubcoreMesh(core_axis_name="c", subcore_axis_name="s")
    run = pl.kernel(
        _body,
        out_type=jax.ShapeDtypeStruct((N_TOTAL, HID), jnp.float32),
        mesh=mesh,
        compiler_params=pltpu.CompilerParams(needs_layout_passes=False),
        scratch_types=[
            pltpu.VMEM((BS,), jnp.int32),
            pltpu.VMEM((CHUNK,), jnp.int32),
            pltpu.VMEM((NCH, RC), jnp.int32),
            pltpu.VMEM((NCH, RC), jnp.int32),
            pltpu.VMEM((NXC, XC), jnp.int32),
            pltpu.VMEM((NXC, XC), jnp.int32),
            pltpu.VMEM((RPT, HID), jnp.float32),
            pltpu.VMEM((RC, HID), jnp.float32),
            pltpu.VMEM_SHARED((TAB,), jnp.int32),
            pltpu.SemaphoreType.DMA,
            pltpu.SemaphoreType.DMA,
            pltpu.SemaphoreType.DMA,
            pltpu.SemaphoreType.DMA,
            pltpu.SemaphoreType.DMA,
            pltpu.SemaphoreType.DMA,
            pltpu.SemaphoreType.DMA,
        ],
    )
    return run(x, n_id, hist_emb)
